# baseline (speedup 1.0000x reference)
import functools

import jax
import jax.numpy as jnp
from jax.experimental import pallas as pl

TARGET_ENTROPY = 8.0
N_BACKGROUND = 4096
COL_BLOCK = 2048


def _simmul_body(x_ref, w_ref, out_ref):
    x = x_ref[...]
    w = w_ref[...]
    wn = w * jax.lax.rsqrt(jnp.sum(w * w, axis=1, keepdims=True))
    out_ref[...] = jax.lax.dot_general(
        x, wn, (((1,), (1,)), ((), ())), preferred_element_type=jnp.float32
    )


def _similarities(norm_points, memory_bank):
    B, D = norm_points.shape
    K, _ = memory_bank.shape
    grid = (pl.cdiv(K, COL_BLOCK),)
    return pl.pallas_call(
        _simmul_body,
        grid=grid,
        in_specs=[
            pl.BlockSpec((B, D), lambda j: (0, 0)),
            pl.BlockSpec((COL_BLOCK, D), lambda j: (j, 0)),
        ],
        out_specs=pl.BlockSpec((B, COL_BLOCK), lambda j: (0, j)),
        out_shape=jax.ShapeDtypeStruct((B, K), jnp.float32),
    )(norm_points, memory_bank)


def _calculate_entropy(similarities, point_indices, t):
    F = jnp.exp(similarities / t[:, None])
    rows = jnp.arange(F.shape[0])
    F = F.at[rows, point_indices].set(0.0)
    r = F / jnp.sum(F, axis=1, keepdims=True)
    return -jnp.sum(r * jnp.log(r + 1e-07), axis=1)


def kernel(points, point_indices, memory_bank):
    norm_points = points / jnp.sqrt(jnp.sum(points**2, axis=1, keepdims=True))
    similarities = _similarities(norm_points, memory_bank)

    hard_similarities, _ = jax.lax.top_k(similarities, N_BACKGROUND)

    left, right = 0.0, 10.0
    scale = (right - left) / 4.0
    centers = jnp.full((similarities.shape[0],), (right + left) / 2.0, jnp.float32)
    for _ in range(13):
        entropy = _calculate_entropy(hard_similarities, point_indices, centers)
        indication = 2.0 * (entropy < TARGET_ENTROPY).astype(jnp.float32) - 1.0
        centers = centers + scale * indication
        scale = scale / 2.0
    entropy = _calculate_entropy(hard_similarities, point_indices, centers)

    rows = jnp.arange(similarities.shape[0])
    positive_similarities = similarities[rows, point_indices]
    condition_p = jnp.exp(positive_similarities / centers - 1.0 / centers) / jnp.sum(
        jnp.exp(hard_similarities / centers[:, None] - 1.0 / centers[:, None]), axis=1
    )
    loss = -jnp.mean(jnp.log(condition_p + 1e-07))
    return loss, similarities, jnp.mean(centers), jnp.mean(entropy)


# speedup vs baseline: 8.5312x; 8.5312x over previous
import jax
import jax.numpy as jnp
from jax.experimental import pallas as pl

_TARGET_ENTROPY = 8.0
_NB = 4096
_B = 1024
_D = 16
_K = 100000
_ROWS = 32
_CHUNK = 2048
_NFULL = _K // _CHUNK
_TAIL0 = _NFULL * _CHUNK
_BITERS = 21
_M = 14
_EPS_H = 4095e-7


def _fused_body(pts_ref, pidx_ref, bank_ref, sims_ref, acc_ref):
    i = pl.program_id(0)
    f32 = jnp.float32

    x = pts_ref[...]
    xn = x * jax.lax.rsqrt(jnp.sum(x * x, axis=1, keepdims=True))

    def _mm(w):
        wn = w * jax.lax.rsqrt(jnp.sum(w * w, axis=0, keepdims=True))
        return jax.lax.dot_general(
            xn, wn, (((1,), (0,)), ((), ())), preferred_element_type=f32
        )

    def _mm_chunk(c, carry):
        sims_ref[:, pl.ds(c * _CHUNK, _CHUNK)] = _mm(
            bank_ref[:, pl.ds(c * _CHUNK, _CHUNK)]
        )
        return carry

    jax.lax.fori_loop(0, _NFULL, _mm_chunk, 0)
    sims_ref[:, _TAIL0:] = _mm(bank_ref[:, _TAIL0:])

    p = pidx_ref[...]
    kt_p = p + 1.0

    def _counts(mid_k, mid_p):
        def body(c, carry):
            ck, cp = carry
            v = sims_ref[:, pl.ds(c * _CHUNK, _CHUNK)]
            ck = ck + jnp.sum(jnp.where(v > mid_k, 1.0, 0.0), axis=1, keepdims=True)
            cp = cp + jnp.sum(jnp.where(v > mid_p, 1.0, 0.0), axis=1, keepdims=True)
            return ck, cp
        z = jnp.zeros((_ROWS, 1), f32)
        ck, cp = jax.lax.fori_loop(0, _NFULL, body, (z, z))
        v = sims_ref[:, _TAIL0:]
        ck = ck + jnp.sum(jnp.where(v > mid_k, 1.0, 0.0), axis=1, keepdims=True)
        cp = cp + jnp.sum(jnp.where(v > mid_p, 1.0, 0.0), axis=1, keepdims=True)
        return ck, cp

    def _bis(_, st):
        lo_k, hi_k, n_k, lo_p, hi_p = st
        mid_k = 0.5 * (lo_k + hi_k)
        mid_p = 0.5 * (lo_p + hi_p)
        ck, cp = _counts(mid_k, mid_p)
        ge_k = ck >= float(_NB)
        n_k = jnp.where(ge_k, ck, n_k)
        lo_k = jnp.where(ge_k, mid_k, lo_k)
        hi_k = jnp.where(ge_k, hi_k, mid_k)
        ge_p = cp >= kt_p
        lo_p = jnp.where(ge_p, mid_p, lo_p)
        hi_p = jnp.where(ge_p, hi_p, mid_p)
        return lo_k, hi_k, n_k, lo_p, hi_p

    ones = jnp.ones((_ROWS, 1), f32)
    st0 = (-1.001 * ones, 1.001 * ones, float(_K) * ones, -1.001 * ones, 1.001 * ones)
    lo_k, _, n_k, v_p, _ = jax.lax.fori_loop(0, _BITERS, _bis, st0)

    def _p1_body(c, s):
        v = sims_ref[:, pl.ds(c * _CHUNK, _CHUNK)]
        return s + jnp.sum(jnp.where(v > lo_k, v, 0.0), axis=1, keepdims=True)

    p1 = jax.lax.fori_loop(0, _NFULL, _p1_body, jnp.zeros((_ROWS, 1), f32))
    v = sims_ref[:, _TAIL0:]
    p1 = p1 + jnp.sum(jnp.where(v > lo_k, v, 0.0), axis=1, keepdims=True)

    excess = n_k - float(_NB)
    vbar = (p1 - excess * lo_k) / float(_NB)

    def _mom(v):
        d = jnp.where(v > lo_k, v - vbar, 0.0)
        cur = d
        out = []
        for m in range(1, _M + 1):
            out.append(jnp.sum(cur, axis=1, keepdims=True))
            if m < _M:
                cur = cur * d
        return tuple(out)

    def _mom_body(c, carry):
        part = _mom(sims_ref[:, pl.ds(c * _CHUNK, _CHUNK)])
        return tuple(a + b for a, b in zip(carry, part))

    z14 = tuple(jnp.zeros((_ROWS, 1), f32) for _ in range(_M))
    cs = jax.lax.fori_loop(0, _NFULL, _mom_body, z14)
    cs = tuple(a + b for a, b in zip(cs, _mom(sims_ref[:, _TAIL0:])))
    dlo = lo_k - vbar
    corr = dlo
    cs_c = [None] * (_M + 1)
    cs_c[0] = float(_NB) * jnp.ones((_ROWS, 1), f32)
    for m in range(1, _M + 1):
        cs_c[m] = cs[m - 1] - excess * corr
        corr = corr * dlo

    def _sm(u):
        t0 = jnp.zeros((_ROWS, 1), f32)
        t1 = jnp.zeros((_ROWS, 1), f32)
        cm = jnp.ones((_ROWS, 1), f32)
        for m in range(_M + 1):
            t0 = t0 + cm * cs_c[m]
            if m < _M:
                t1 = t1 + cm * cs_c[m + 1]
            cm = cm * u / float(m + 1)
        e = jnp.exp(u * vbar)
        return e * t0, e * (vbar * t0 + t1)

    def _entropy(u):
        s, mv = _sm(u)
        ep = jnp.exp(u * v_p)
        sp = s - ep
        mp = mv - v_p * ep
        return jnp.log(sp) - u * mp / sp - _EPS_H

    centers = 5.0 * jnp.ones((_ROWS, 1), f32)
    scale = 2.5
    for _ in range(13):
        h = _entropy(1.0 / centers)
        ind = 2.0 * jnp.where(h < _TARGET_ENTROPY, 1.0, 0.0) - 1.0
        centers = centers + scale * ind
        scale = scale * 0.5
    u_f = 1.0 / centers
    h_f = _entropy(u_f)

    pos = jnp.zeros((_ROWS, 1), f32)
    for j in range(_NB // _CHUNK):
        v = sims_ref[:, j * _CHUNK:(j + 1) * _CHUNK]
        lane = jax.lax.broadcasted_iota(jnp.int32, (_ROWS, _CHUNK), 1).astype(f32)
        pos = pos + jnp.sum(
            jnp.where(lane == (p - float(j * _CHUNK)), v, 0.0),
            axis=1, keepdims=True,
        )

    s_f, _ = _sm(u_f)
    denom = jnp.exp(-u_f) * s_f
    cond = jnp.exp((pos - 1.0) * u_f) / denom
    ll = jnp.log(cond + 1e-7)

    vec = jnp.concatenate(
        [jnp.sum(ll, axis=0, keepdims=True),
         jnp.sum(centers, axis=0, keepdims=True),
         jnp.sum(h_f, axis=0, keepdims=True)], axis=1)
    acc_ref[...] = jnp.where(i == 0, vec, acc_ref[...] + vec)


def kernel(points, point_indices, memory_bank):
    bank_t = memory_bank.T
    pidx_f = point_indices.astype(jnp.float32).reshape(_B, 1)
    sims, acc = pl.pallas_call(
        _fused_body,
        grid=(_B // _ROWS,),
        in_specs=[
            pl.BlockSpec((_ROWS, _D), lambda i: (i, 0)),
            pl.BlockSpec((_ROWS, 1), lambda i: (i, 0)),
            pl.BlockSpec((_D, _K), lambda i: (0, 0)),
        ],
        out_specs=[
            pl.BlockSpec((_ROWS, _K), lambda i: (i, 0)),
            pl.BlockSpec((1, 3), lambda i: (0, 0)),
        ],
        out_shape=[
            jax.ShapeDtypeStruct((_B, _K), jnp.float32),
            jax.ShapeDtypeStruct((1, 3), jnp.float32),
        ],
    )(points, pidx_f, bank_t)
    inv_b = 1.0 / float(_B)
    loss = -acc[0, 0] * inv_b
    return loss, sims, acc[0, 1] * inv_b, acc[0, 2] * inv_b
